# baseline (speedup 1.0000x reference)
import functools
import math

import jax
import jax.numpy as jnp
from jax import lax
from jax.experimental import pallas as pl
from jax.experimental.pallas import tpu as pltpu
from jax.experimental.pallas import tpu_sc as plsc

VOCAB = 1000000
D = 64
B_TOTAL = 4096 * 200
GROUP = 128
NW = 32
GROUPS_TOTAL = B_TOTAL // GROUP
GROUPS_PER_W = GROUPS_TOTAL // NW
K = 8
CHUNKS = GROUPS_PER_W // K
ROWS_PER_CHUNK = K * GROUP
SCALE = math.sqrt(D)


def _emb_kernel(idx_hbm, tab_hbm, out_hbm, idx_v, rows_v, sem):
    wid = lax.axis_index("s") * 2 + lax.axis_index("c")
    g0 = wid * GROUPS_PER_W

    def chunk_body(c, carry):
        gbase = g0 + c * K
        pltpu.sync_copy(idx_hbm.at[pl.ds(gbase, K)], idx_v)
        copies = [
            pltpu.async_copy(
                tab_hbm.at[idx_v.at[j]],
                rows_v.at[pl.ds(j * GROUP, GROUP)],
                sem,
            )
            for j in range(K)
        ]
        for cp in copies:
            cp.wait()

        def scale_body(i, carry2):
            for r in range(8):
                for jj in range(D // 16):
                    sl = pl.ds(jj * 16, 16)
                    rows_v[i * 8 + r, sl] = rows_v[i * 8 + r, sl] * SCALE
            return carry2

        lax.fori_loop(0, ROWS_PER_CHUNK // 8, scale_body, 0, unroll=False)

        pltpu.sync_copy(rows_v, out_hbm.at[pl.ds(gbase * GROUP, ROWS_PER_CHUNK)])
        return carry

    lax.fori_loop(0, CHUNKS, chunk_body, 0, unroll=False)


@jax.jit
def kernel(token_indices, embedding_weight):
    idx = token_indices.reshape(GROUPS_TOTAL, GROUP)
    mesh = plsc.VectorSubcoreMesh(core_axis_name="c", subcore_axis_name="s")
    out = pl.kernel(
        _emb_kernel,
        mesh=mesh,
        out_type=jax.ShapeDtypeStruct((B_TOTAL, D), jnp.float32),
        scratch_types=[
            pltpu.VMEM((K, GROUP), jnp.int32),
            pltpu.VMEM((ROWS_PER_CHUNK, D), jnp.float32),
            pltpu.SemaphoreType.DMA,
        ],
        compiler_params=pltpu.CompilerParams(use_tc_tiling_on_sc=False),
    )(idx, embedding_weight)
    return out.reshape(token_indices.shape[0], token_indices.shape[1], D)
